# baseline (speedup 1.0000x reference)
import jax
import jax.numpy as jnp
from jax.experimental import pallas as pl
from jax.experimental.pallas import tpu as pltpu

_B, _T, _D_IN, _D_SAE, _K = 256, 4, 768, 16384, 128
_D_FLAT = _T * _D_IN

_ENC_MT = 1024
_SEL_BT = 64
_DEC_KT = 1024
_M_W = 512

_N_ENC = _D_SAE // _ENC_MT
_N_SEL = _B // _SEL_BT
_N_KT = _D_SAE // _DEC_KT
_N_DEC = _T * _N_KT
_SEL0 = _N_ENC
_DEC0 = _N_ENC + _N_SEL
_GRID = _DEC0 + _N_DEC

_MSEARCH_BITS = 16


def _body(x_ref, we_ref, we2_ref, be_ref, wd_ref, bd_ref, z_ref, xh_ref, scr_ref,
          m_ref, thrf_ref, lo_ref, hi_ref, thr_ref, fnd_ref):
    i = pl.program_id(0)

    @pl.when(i < _SEL0)
    def _encode():
        acc = (jnp.dot(x_ref[:, :_D_FLAT // 2], we_ref[...],
                       preferred_element_type=jnp.float32)
               + jnp.dot(x_ref[:, _D_FLAT // 2:], we2_ref[...],
                         preferred_element_type=jnp.float32))
        r = jnp.maximum(acc + be_ref[...], 0.0)
        scr_ref[:, pl.ds(i * _ENC_MT, _ENC_MT)] = r
        rmax = jnp.maximum(r[:, :_M_W], r[:, _M_W:])

        @pl.when(i == 0)
        def _minit():
            m_ref[...] = rmax

        @pl.when(i > 0)
        def _mupd():
            m_ref[...] = jnp.maximum(m_ref[...], rmax)

    @pl.when((i >= _SEL0) & (i < _DEC0))
    def _select():
        rb = i - _SEL0
        v = scr_ref[pl.ds(rb * _SEL_BT, _SEL_BT), :]
        bits = jax.lax.bitcast_convert_type(v, jnp.int32)
        bm = jax.lax.bitcast_convert_type(
            m_ref[pl.ds(rb * _SEL_BT, _SEL_BT), :], jnp.int32)

        def mstep(j, lo):
            cand = lo | (1 << (30 - j))
            cnt = jnp.sum((bm >= cand).astype(jnp.int32), axis=1,
                          keepdims=True)
            return jnp.where(cnt >= _K, cand, lo)

        lo0 = jax.lax.fori_loop(0, _MSEARCH_BITS, mstep,
                                jnp.zeros((_SEL_BT, 1), jnp.int32))
        hi0 = jnp.max(bm, axis=1, keepdims=True) + 1

        zeros = jnp.zeros((_SEL_BT, 1), jnp.int32)

        def halve(lo, hi, thr, fnd):
            mid = jax.lax.shift_right_logical(lo + hi, 1)
            cnt = jnp.sum((bits >= mid).astype(jnp.int32), axis=1,
                          keepdims=True)
            exact = (cnt == _K) & (fnd == 0)
            thr = jnp.where(exact, mid, thr)
            fnd = jnp.where(exact, 1, fnd)
            live = fnd == 0
            ge = cnt >= _K
            lo = jnp.where(live & ge, mid, lo)
            hi = jnp.where(live & ~ge, mid, hi)
            return lo, hi, thr, fnd

        def pstep(j, c):
            return halve(*c)

        lo1, hi1, thr1, fnd1 = jax.lax.fori_loop(
            0, 10, pstep, (lo0, hi0, zeros, zeros))

        lo_ref[...] = lo1
        hi_ref[...] = hi1
        thr_ref[...] = thr1
        fnd_ref[...] = fnd1

        def cond(c):
            it, done = c
            return (it < 12) & (done == 0)

        def step(c):
            it, _ = c
            lo, hi, thr, fnd = (lo_ref[...], hi_ref[...], thr_ref[...],
                                fnd_ref[...])
            for _ in range(2):
                lo, hi, thr, fnd = halve(lo, hi, thr, fnd)
            lo_ref[...] = lo
            hi_ref[...] = hi
            thr_ref[...] = thr
            fnd_ref[...] = fnd
            done = jnp.all((fnd == 1) | (hi - lo <= 1))
            return (it + 1, done.astype(jnp.int32))

        jax.lax.while_loop(cond, step, (0, 0))
        thr = jnp.where(fnd_ref[...] == 1, thr_ref[...], lo_ref[...])
        thrf_ref[pl.ds(rb * _SEL_BT, _SEL_BT), :] = (
            jax.lax.bitcast_convert_type(thr, jnp.float32))

    @pl.when(i >= _DEC0)
    def _decode():
        j = i - _DEC0
        k = j % _N_KT

        @pl.when(k == 0)
        def _init():
            xh_ref[...] = jnp.broadcast_to(bd_ref[0], xh_ref.shape)


        @pl.when(j < _N_KT)
        def _mask():
            vt = scr_ref[:, pl.ds(k * _DEC_KT, _DEC_KT)]
            zt = jnp.where(vt >= thrf_ref[...], vt, 0.0)
            z_ref[...] = zt
            scr_ref[:, pl.ds(k * _DEC_KT, _DEC_KT)] = zt

        zt = scr_ref[:, pl.ds(k * _DEC_KT, _DEC_KT)]
        acc = jnp.dot(zt, wd_ref[0], preferred_element_type=jnp.float32)
        xh_ref[...] += acc


def _we_map(i):
    return (0, jnp.minimum(i, _N_ENC - 1))


def _z_map(i):
    return (0, jnp.clip(i - _DEC0, 0, _N_KT - 1))


def _dec_t(i):
    return jnp.clip((i - _DEC0) // _N_KT, 0, _T - 1)


def kernel(x, W_enc, b_enc, W_dec, b_dec):
    x2 = x.reshape(_B, _D_FLAT)
    w_enc2 = W_enc.reshape(_D_FLAT, _D_SAE)
    b_enc2 = b_enc.reshape(1, _D_SAE)
    b_dec2 = b_dec.reshape(_T, 1, _D_IN)

    z, x_hat = pl.pallas_call(
        _body,
        grid=(_GRID,),
        in_specs=[
            pl.BlockSpec((_B, _D_FLAT), lambda i: (0, 0)),
            pl.BlockSpec((_D_FLAT // 2, _ENC_MT), _we_map),
            pl.BlockSpec((_D_FLAT // 2, _ENC_MT), _we_map),
            pl.BlockSpec((1, _ENC_MT), _we_map),
            pl.BlockSpec((1, _DEC_KT, _D_IN),
                         lambda i: (_dec_t(i),
                                    jnp.clip(i - _DEC0, 0, _N_DEC - 1)
                                    % _N_KT,
                                    0)),
            pl.BlockSpec((1, 1, _D_IN), lambda i: (_dec_t(i), 0, 0)),
        ],
        out_specs=[
            pl.BlockSpec((_B, _DEC_KT), _z_map),
            pl.BlockSpec((_B, _D_IN), lambda i: (0, _dec_t(i))),
        ],
        out_shape=[
            jax.ShapeDtypeStruct((_B, _D_SAE), jnp.float32),
            jax.ShapeDtypeStruct((_B, _D_FLAT), jnp.float32),
        ],
        scratch_shapes=[
            pltpu.VMEM((_B, _D_SAE), jnp.float32),
            pltpu.VMEM((_B, _M_W), jnp.float32),
            pltpu.VMEM((_B, 1), jnp.float32),
            pltpu.VMEM((_SEL_BT, 1), jnp.int32),
            pltpu.VMEM((_SEL_BT, 1), jnp.int32),
            pltpu.VMEM((_SEL_BT, 1), jnp.int32),
            pltpu.VMEM((_SEL_BT, 1), jnp.int32),
        ],
        compiler_params=pltpu.CompilerParams(
            dimension_semantics=("arbitrary",),
            vmem_limit_bytes=120 * 1024 * 1024,
        ),
    )(x2, w_enc2[:_D_FLAT // 2], w_enc2[_D_FLAT // 2:], b_enc2, W_dec,
      b_dec2)

    return (x_hat.reshape(_B, _T, _D_IN), z)


# speedup vs baseline: 1.5743x; 1.5743x over previous
import jax
import jax.numpy as jnp
from jax.experimental import pallas as pl
from jax.experimental.pallas import tpu as pltpu

_B, _T, _D_IN, _D_SAE, _K = 256, 4, 768, 16384, 128
_D_FLAT = _T * _D_IN

_ENC_MT = 1024
_SEL_BT = 64
_DEC_KT = 1024
_M_W = 512

_N_ENC = _D_SAE // _ENC_MT
_N_SEL = _B // _SEL_BT
_N_KT = _D_SAE // _DEC_KT
_N_DEC = _T * _N_KT
_SEL0 = _N_ENC
_DEC0 = _N_ENC + _N_SEL
_GRID = _DEC0 + _N_DEC

_MSEARCH_BITS = 16


def _body(x_ref, we_ref, we2_ref, be_ref, wd_ref, bd_ref, z_ref, xh_ref, scr_ref,
          m_ref, thrf_ref, lo_ref, hi_ref, thr_ref, fnd_ref):
    i = pl.program_id(0)

    @pl.when(i < _SEL0)
    def _encode():
        acc = (jnp.dot(x_ref[:, :_D_FLAT // 2], we_ref[...],
                       preferred_element_type=jnp.float32)
               + jnp.dot(x_ref[:, _D_FLAT // 2:], we2_ref[...],
                         preferred_element_type=jnp.float32))
        r = jnp.maximum(acc + be_ref[...], 0.0)
        scr_ref[:, pl.ds(i * _ENC_MT, _ENC_MT)] = r
        rmax = jnp.maximum(r[:, :_M_W], r[:, _M_W:])

        @pl.when(i == 0)
        def _minit():
            m_ref[...] = rmax

        @pl.when(i > 0)
        def _mupd():
            m_ref[...] = jnp.maximum(m_ref[...], rmax)

    @pl.when((i >= _SEL0) & (i < _DEC0))
    def _select():
        rb = i - _SEL0
        v = scr_ref[pl.ds(rb * _SEL_BT, _SEL_BT), :]
        bits = jax.lax.bitcast_convert_type(v, jnp.int32)
        bm = jax.lax.bitcast_convert_type(
            m_ref[pl.ds(rb * _SEL_BT, _SEL_BT), :], jnp.int32)

        def mstep(j, lo):
            cand = lo | (1 << (30 - j))
            cnt = jnp.sum((bm >= cand).astype(jnp.int32), axis=1,
                          keepdims=True)
            return jnp.where(cnt >= _K, cand, lo)

        lo0 = jax.lax.fori_loop(0, _MSEARCH_BITS, mstep,
                                jnp.zeros((_SEL_BT, 1), jnp.int32))
        hi0 = jnp.max(bm, axis=1, keepdims=True) + 1

        zeros = jnp.zeros((_SEL_BT, 1), jnp.int32)

        def halve(lo, hi, thr, fnd):
            mid = jax.lax.shift_right_logical(lo + hi, 1)
            cnt = jnp.sum((bits >= mid).astype(jnp.int32), axis=1,
                          keepdims=True)
            exact = (cnt == _K) & (fnd == 0)
            thr = jnp.where(exact, mid, thr)
            fnd = jnp.where(exact, 1, fnd)
            live = fnd == 0
            ge = cnt >= _K
            lo = jnp.where(live & ge, mid, lo)
            hi = jnp.where(live & ~ge, mid, hi)
            return lo, hi, thr, fnd

        def pstep(j, c):
            return halve(*c)

        lo1, hi1, thr1, fnd1 = jax.lax.fori_loop(
            0, 10, pstep, (lo0, hi0, zeros, zeros))

        lo_ref[...] = lo1
        hi_ref[...] = hi1
        thr_ref[...] = thr1
        fnd_ref[...] = fnd1

        def cond(c):
            it, done = c
            return (it < 12) & (done == 0)

        def step(c):
            it, _ = c
            lo, hi, thr, fnd = (lo_ref[...], hi_ref[...], thr_ref[...],
                                fnd_ref[...])
            for _ in range(2):
                lo, hi, thr, fnd = halve(lo, hi, thr, fnd)
            lo_ref[...] = lo
            hi_ref[...] = hi
            thr_ref[...] = thr
            fnd_ref[...] = fnd
            done = jnp.all((fnd == 1) | (hi - lo <= 1))
            return (it + 1, done.astype(jnp.int32))

        jax.lax.while_loop(cond, step, (0, 0))
        thr = jnp.where(fnd_ref[...] == 1, thr_ref[...], lo_ref[...])
        thrf_ref[pl.ds(rb * _SEL_BT, _SEL_BT), :] = (
            jax.lax.bitcast_convert_type(thr, jnp.float32))

    @pl.when(i >= _DEC0)
    def _decode():
        j = i - _DEC0
        k = j % _N_KT

        @pl.when(k == 0)
        def _init():
            xh_ref[...] = jnp.broadcast_to(bd_ref[0], xh_ref.shape)


        @pl.when(j < _N_KT)
        def _mask():
            vt = scr_ref[:, pl.ds(k * _DEC_KT, _DEC_KT)]
            zt = jnp.where(vt >= thrf_ref[...], vt, 0.0)
            z_ref[...] = zt
            scr_ref[:, pl.ds(k * _DEC_KT, _DEC_KT)] = zt

        zt = scr_ref[:, pl.ds(k * _DEC_KT, _DEC_KT)]
        acc = jnp.dot(zt, wd_ref[0], preferred_element_type=jnp.float32)
        xh_ref[...] += acc


def _we_map(i):
    return (0, jnp.minimum(i, _N_ENC - 1))


def _z_map(i):
    return (0, jnp.clip(i - _DEC0, 0, _N_KT - 1))


def _dec_t(i):
    return jnp.clip((i - _DEC0) // _N_KT, 0, _T - 1)


def kernel(x, W_enc, b_enc, W_dec, b_dec):
    x2 = x.reshape(_B, _D_FLAT)
    w_enc2 = W_enc.reshape(_D_FLAT, _D_SAE)
    b_enc2 = b_enc.reshape(1, _D_SAE)
    b_dec2 = b_dec.reshape(_T, 1, _D_IN)

    z, x_hat = pl.pallas_call(
        _body,
        grid=(_GRID,),
        in_specs=[
            pl.BlockSpec((_B, _D_FLAT), lambda i: (0, 0)),
            pl.BlockSpec((_D_FLAT // 2, _ENC_MT), _we_map),
            pl.BlockSpec((_D_FLAT // 2, _ENC_MT),
                         lambda i: (1, jnp.minimum(i, _N_ENC - 1))),
            pl.BlockSpec((1, _ENC_MT), _we_map),
            pl.BlockSpec((1, _DEC_KT, _D_IN),
                         lambda i: (_dec_t(i),
                                    jnp.clip(i - _DEC0, 0, _N_DEC - 1)
                                    % _N_KT,
                                    0)),
            pl.BlockSpec((1, 1, _D_IN), lambda i: (_dec_t(i), 0, 0)),
        ],
        out_specs=[
            pl.BlockSpec((_B, _DEC_KT), _z_map),
            pl.BlockSpec((_B, _D_IN), lambda i: (0, _dec_t(i))),
        ],
        out_shape=[
            jax.ShapeDtypeStruct((_B, _D_SAE), jnp.float32),
            jax.ShapeDtypeStruct((_B, _D_FLAT), jnp.float32),
        ],
        scratch_shapes=[
            pltpu.VMEM((_B, _D_SAE), jnp.float32),
            pltpu.VMEM((_B, _M_W), jnp.float32),
            pltpu.VMEM((_B, 1), jnp.float32),
            pltpu.VMEM((_SEL_BT, 1), jnp.int32),
            pltpu.VMEM((_SEL_BT, 1), jnp.int32),
            pltpu.VMEM((_SEL_BT, 1), jnp.int32),
            pltpu.VMEM((_SEL_BT, 1), jnp.int32),
        ],
        compiler_params=pltpu.CompilerParams(
            dimension_semantics=("arbitrary",),
            vmem_limit_bytes=120 * 1024 * 1024,
        ),
    )(x2, w_enc2, w_enc2, b_enc2, W_dec, b_dec2)

    return (x_hat.reshape(_B, _T, _D_IN), z)


# speedup vs baseline: 1.6284x; 1.0344x over previous
import jax
import jax.numpy as jnp
from jax.experimental import pallas as pl
from jax.experimental.pallas import tpu as pltpu

_B, _T, _D_IN, _D_SAE, _K = 256, 4, 768, 16384, 128
_D_FLAT = _T * _D_IN

_ENC_MT = 1024
_SEL_BT = 64
_DEC_KT = 1024
_M_W = 512

_N_ENC = _D_SAE // _ENC_MT
_N_SEL = _B // _SEL_BT
_N_KT = _D_SAE // _DEC_KT
_N_DEC = _T * _N_KT
_SEL0 = _N_ENC
_DEC0 = _N_ENC + _N_SEL
_GRID = _DEC0 + _N_DEC

def _body(x_ref, we_ref, be_ref, wd_ref, bd_ref, z_ref, xh_ref, scr_ref,
          m_ref, thrf_ref, lo_ref, hi_ref, thr_ref, fnd_ref):
    i = pl.program_id(0)

    @pl.when(i < _SEL0)
    def _encode():
        acc = jnp.dot(x_ref[...], we_ref[...],
                      preferred_element_type=jnp.float32)
        r = jnp.maximum(acc + be_ref[...], 0.0)
        scr_ref[:, pl.ds(i * _ENC_MT, _ENC_MT)] = r
        rmax = jnp.maximum(r[:, :_M_W], r[:, _M_W:])

        @pl.when(i == 0)
        def _minit():
            m_ref[...] = rmax

        @pl.when(i > 0)
        def _mupd():
            m_ref[...] = jnp.maximum(m_ref[...], rmax)

    @pl.when((i >= _SEL0) & (i < _DEC0))
    def _select():
        rb = i - _SEL0
        v = scr_ref[pl.ds(rb * _SEL_BT, _SEL_BT), :]
        bits = jax.lax.bitcast_convert_type(v, jnp.int32)
        mrows = m_ref[pl.ds(rb * _SEL_BT, _SEL_BT), :]
        m2 = jnp.maximum(jnp.maximum(mrows[:, :128], mrows[:, 128:256]),
                         jnp.maximum(mrows[:, 256:384], mrows[:, 384:]))
        lo0 = jax.lax.bitcast_convert_type(
            jnp.min(m2, axis=1, keepdims=True), jnp.int32)
        hi0 = jax.lax.bitcast_convert_type(
            jnp.max(m2, axis=1, keepdims=True), jnp.int32) + 1

        zeros = jnp.zeros((_SEL_BT, 1), jnp.int32)

        def halve(lo, hi, thr, fnd):
            mid = jax.lax.shift_right_logical(lo + hi, 1)
            cnt = jnp.sum((bits >= mid).astype(jnp.int32), axis=1,
                          keepdims=True)
            exact = (cnt == _K) & (fnd == 0)
            thr = jnp.where(exact, mid, thr)
            fnd = jnp.where(exact, 1, fnd)
            live = fnd == 0
            ge = cnt >= _K
            lo = jnp.where(live & ge, mid, lo)
            hi = jnp.where(live & ~ge, mid, hi)
            return lo, hi, thr, fnd

        def pstep(j, c):
            return halve(*c)

        lo1, hi1, thr1, fnd1 = jax.lax.fori_loop(
            0, 10, pstep, (lo0, hi0, zeros, zeros))

        lo_ref[...] = lo1
        hi_ref[...] = hi1
        thr_ref[...] = thr1
        fnd_ref[...] = fnd1

        def cond(c):
            it, done = c
            return (it < 12) & (done == 0)

        def step(c):
            it, _ = c
            lo, hi, thr, fnd = (lo_ref[...], hi_ref[...], thr_ref[...],
                                fnd_ref[...])
            for _ in range(2):
                lo, hi, thr, fnd = halve(lo, hi, thr, fnd)
            lo_ref[...] = lo
            hi_ref[...] = hi
            thr_ref[...] = thr
            fnd_ref[...] = fnd
            done = jnp.all((fnd == 1) | (hi - lo <= 1))
            return (it + 1, done.astype(jnp.int32))

        jax.lax.while_loop(cond, step, (0, 0))
        thr = jnp.where(fnd_ref[...] == 1, thr_ref[...], lo_ref[...])
        thrf_ref[pl.ds(rb * _SEL_BT, _SEL_BT), :] = (
            jax.lax.bitcast_convert_type(thr, jnp.float32))

    @pl.when(i >= _DEC0)
    def _decode():
        j = i - _DEC0
        k = j % _N_KT

        @pl.when(k == 0)
        def _init():
            xh_ref[...] = jnp.broadcast_to(bd_ref[0], xh_ref.shape)


        @pl.when(j < _N_KT)
        def _mask():
            vt = scr_ref[:, pl.ds(k * _DEC_KT, _DEC_KT)]
            zt = jnp.where(vt >= thrf_ref[...], vt, 0.0)
            z_ref[...] = zt
            scr_ref[:, pl.ds(k * _DEC_KT, _DEC_KT)] = zt

        zt = scr_ref[:, pl.ds(k * _DEC_KT, _DEC_KT)]
        acc = jnp.dot(zt, wd_ref[0], preferred_element_type=jnp.float32)
        xh_ref[...] += acc


def _we_map(i):
    return (0, jnp.minimum(i, _N_ENC - 1))


def _z_map(i):
    return (0, jnp.clip(i - _DEC0, 0, _N_KT - 1))


def _dec_t(i):
    return jnp.clip((i - _DEC0) // _N_KT, 0, _T - 1)


def kernel(x, W_enc, b_enc, W_dec, b_dec):
    x2 = x.reshape(_B, _D_FLAT)
    w_enc2 = W_enc.reshape(_D_FLAT, _D_SAE)
    b_enc2 = b_enc.reshape(1, _D_SAE)
    b_dec2 = b_dec.reshape(_T, 1, _D_IN)

    z, x_hat = pl.pallas_call(
        _body,
        grid=(_GRID,),
        in_specs=[
            pl.BlockSpec((_B, _D_FLAT), lambda i: (0, 0)),
            pl.BlockSpec((_D_FLAT, _ENC_MT), _we_map),
            pl.BlockSpec((1, _ENC_MT), _we_map),
            pl.BlockSpec((1, _DEC_KT, _D_IN),
                         lambda i: (_dec_t(i),
                                    jnp.clip(i - _DEC0, 0, _N_DEC - 1)
                                    % _N_KT,
                                    0)),
            pl.BlockSpec((1, 1, _D_IN), lambda i: (_dec_t(i), 0, 0)),
        ],
        out_specs=[
            pl.BlockSpec((_B, _DEC_KT), _z_map),
            pl.BlockSpec((_B, _D_IN), lambda i: (0, _dec_t(i))),
        ],
        out_shape=[
            jax.ShapeDtypeStruct((_B, _D_SAE), jnp.float32),
            jax.ShapeDtypeStruct((_B, _D_FLAT), jnp.float32),
        ],
        scratch_shapes=[
            pltpu.VMEM((_B, _D_SAE), jnp.float32),
            pltpu.VMEM((_B, _M_W), jnp.float32),
            pltpu.VMEM((_B, 1), jnp.float32),
            pltpu.VMEM((_SEL_BT, 1), jnp.int32),
            pltpu.VMEM((_SEL_BT, 1), jnp.int32),
            pltpu.VMEM((_SEL_BT, 1), jnp.int32),
            pltpu.VMEM((_SEL_BT, 1), jnp.int32),
        ],
        compiler_params=pltpu.CompilerParams(
            dimension_semantics=("arbitrary",),
            vmem_limit_bytes=120 * 1024 * 1024,
        ),
    )(x2, w_enc2, b_enc2, W_dec, b_dec2)

    return (x_hat.reshape(_B, _T, _D_IN), z)


# speedup vs baseline: 1.6354x; 1.0043x over previous
import jax
import jax.numpy as jnp
from jax.experimental import pallas as pl
from jax.experimental.pallas import tpu as pltpu

_B, _T, _D_IN, _D_SAE, _K = 256, 4, 768, 16384, 128
_D_FLAT = _T * _D_IN

_ENC_MT = 1024
_SEL_BT = 64
_DEC_KT = 1024
_M_W = 512

_N_ENC = _D_SAE // _ENC_MT
_N_SEL = _B // _SEL_BT
_N_KT = _D_SAE // _DEC_KT
_N_DEC = _T * _N_KT
_SEL0 = _N_ENC
_DEC0 = _N_ENC + _N_SEL
_GRID = _DEC0 + _N_DEC

def _body(x_ref, we_ref, be_ref, wd_ref, bd_ref, z_ref, xh_ref, scr_ref,
          m_ref, thrf_ref, lo_ref, hi_ref, thr_ref, fnd_ref):
    i = pl.program_id(0)

    @pl.when(i < _SEL0)
    def _encode():
        acc = jnp.dot(x_ref[...], we_ref[...],
                      preferred_element_type=jnp.float32)
        r = jnp.maximum(acc + be_ref[...], 0.0)
        scr_ref[:, pl.ds(i * _ENC_MT, _ENC_MT)] = r
        rmax = jnp.maximum(r[:, :_M_W], r[:, _M_W:])

        @pl.when(i == 0)
        def _minit():
            m_ref[...] = rmax

        @pl.when(i > 0)
        def _mupd():
            m_ref[...] = jnp.maximum(m_ref[...], rmax)

    @pl.when((i >= _SEL0) & (i < _DEC0))
    def _select():
        rb = i - _SEL0
        v = scr_ref[pl.ds(rb * _SEL_BT, _SEL_BT), :]
        bits = jax.lax.bitcast_convert_type(v, jnp.int32)
        mrows = m_ref[pl.ds(rb * _SEL_BT, _SEL_BT), :]
        m2 = jnp.maximum(jnp.maximum(mrows[:, :128], mrows[:, 128:256]),
                         jnp.maximum(mrows[:, 256:384], mrows[:, 384:]))
        lo0 = jax.lax.bitcast_convert_type(
            jnp.min(m2, axis=1, keepdims=True), jnp.int32)
        hi0 = jax.lax.bitcast_convert_type(
            jnp.max(m2, axis=1, keepdims=True), jnp.int32) + 1

        zeros = jnp.zeros((_SEL_BT, 1), jnp.int32)

        def halve(lo, hi, thr, fnd):
            mid = jax.lax.shift_right_logical(lo + hi, 1)
            cnt = jnp.sum((bits >= mid).astype(jnp.int32), axis=1,
                          keepdims=True)
            exact = (cnt == _K) & (fnd == 0)
            thr = jnp.where(exact, mid, thr)
            fnd = jnp.where(exact, 1, fnd)
            live = fnd == 0
            ge = cnt >= _K
            lo = jnp.where(live & ge, mid, lo)
            hi = jnp.where(live & ~ge, mid, hi)
            return lo, hi, thr, fnd

        def pstep(j, c):
            return halve(*c)

        lo1, hi1, thr1, fnd1 = jax.lax.fori_loop(
            0, 12, pstep, (lo0, hi0, zeros, zeros))

        lo_ref[...] = lo1
        hi_ref[...] = hi1
        thr_ref[...] = thr1
        fnd_ref[...] = fnd1

        def cond(c):
            it, done = c
            return (it < 12) & (done == 0)

        def step(c):
            it, _ = c
            lo, hi, thr, fnd = (lo_ref[...], hi_ref[...], thr_ref[...],
                                fnd_ref[...])
            for _ in range(2):
                lo, hi, thr, fnd = halve(lo, hi, thr, fnd)
            lo_ref[...] = lo
            hi_ref[...] = hi
            thr_ref[...] = thr
            fnd_ref[...] = fnd
            done = jnp.all((fnd == 1) | (hi - lo <= 1))
            return (it + 1, done.astype(jnp.int32))

        jax.lax.while_loop(cond, step, (0, 0))
        thr = jnp.where(fnd_ref[...] == 1, thr_ref[...], lo_ref[...])
        thrf_ref[pl.ds(rb * _SEL_BT, _SEL_BT), :] = (
            jax.lax.bitcast_convert_type(thr, jnp.float32))

    @pl.when(i >= _DEC0)
    def _decode():
        j = i - _DEC0
        k = j % _N_KT

        @pl.when(k == 0)
        def _init():
            xh_ref[...] = jnp.broadcast_to(bd_ref[0], xh_ref.shape)


        @pl.when(j < _N_KT)
        def _mask():
            vt = scr_ref[:, pl.ds(k * _DEC_KT, _DEC_KT)]
            zt = jnp.where(vt >= thrf_ref[...], vt, 0.0)
            z_ref[...] = zt
            scr_ref[:, pl.ds(k * _DEC_KT, _DEC_KT)] = zt

        zt = scr_ref[:, pl.ds(k * _DEC_KT, _DEC_KT)]
        acc = jnp.dot(zt, wd_ref[0], preferred_element_type=jnp.float32)
        xh_ref[...] += acc


def _we_map(i):
    return (0, jnp.minimum(i, _N_ENC - 1))


def _z_map(i):
    return (0, jnp.clip(i - _DEC0, 0, _N_KT - 1))


def _dec_t(i):
    return jnp.clip((i - _DEC0) // _N_KT, 0, _T - 1)


def kernel(x, W_enc, b_enc, W_dec, b_dec):
    x2 = x.reshape(_B, _D_FLAT)
    w_enc2 = W_enc.reshape(_D_FLAT, _D_SAE)
    b_enc2 = b_enc.reshape(1, _D_SAE)
    b_dec2 = b_dec.reshape(_T, 1, _D_IN)

    z, x_hat = pl.pallas_call(
        _body,
        grid=(_GRID,),
        in_specs=[
            pl.BlockSpec((_B, _D_FLAT), lambda i: (0, 0)),
            pl.BlockSpec((_D_FLAT, _ENC_MT), _we_map),
            pl.BlockSpec((1, _ENC_MT), _we_map),
            pl.BlockSpec((1, _DEC_KT, _D_IN),
                         lambda i: (_dec_t(i),
                                    jnp.clip(i - _DEC0, 0, _N_DEC - 1)
                                    % _N_KT,
                                    0)),
            pl.BlockSpec((1, 1, _D_IN), lambda i: (_dec_t(i), 0, 0)),
        ],
        out_specs=[
            pl.BlockSpec((_B, _DEC_KT), _z_map),
            pl.BlockSpec((_B, _D_IN), lambda i: (0, _dec_t(i))),
        ],
        out_shape=[
            jax.ShapeDtypeStruct((_B, _D_SAE), jnp.float32),
            jax.ShapeDtypeStruct((_B, _D_FLAT), jnp.float32),
        ],
        scratch_shapes=[
            pltpu.VMEM((_B, _D_SAE), jnp.float32),
            pltpu.VMEM((_B, _M_W), jnp.float32),
            pltpu.VMEM((_B, 1), jnp.float32),
            pltpu.VMEM((_SEL_BT, 1), jnp.int32),
            pltpu.VMEM((_SEL_BT, 1), jnp.int32),
            pltpu.VMEM((_SEL_BT, 1), jnp.int32),
            pltpu.VMEM((_SEL_BT, 1), jnp.int32),
        ],
        compiler_params=pltpu.CompilerParams(
            dimension_semantics=("arbitrary",),
            vmem_limit_bytes=120 * 1024 * 1024,
        ),
    )(x2, w_enc2, b_enc2, W_dec, b_dec2)

    return (x_hat.reshape(_B, _T, _D_IN), z)
